# Initial kernel scaffold; baseline (speedup 1.0000x reference)
import jax
import jax.numpy as jnp
from jax.experimental import pallas as pl


def kernel(x):
    raise NotImplementedError("write your pallas kernel here")


# speedup vs baseline: 11.2084x; 11.2084x over previous
import functools

import jax
import jax.numpy as jnp
from jax.experimental import pallas as pl

TOPK_FRAC = 0.1


def _select_mask_kernel(x_ref, o_ref, *, k: int, n_iters: int = 31):
    xv = x_ref[0]
    bits = jax.lax.bitcast_convert_type(xv, jnp.int32) & jnp.int32(0x7FFFFFFF)

    def body(_, lohi):
        lo, hi = lohi
        mid = lo + ((hi - lo) >> 1)
        cnt = jnp.sum((bits >= mid).astype(jnp.int32))
        ge = cnt >= k
        return (jnp.where(ge, mid, lo), jnp.where(ge, hi, mid))

    lo, _ = jax.lax.fori_loop(
        0, n_iters, body, (jnp.int32(0), jnp.int32(0x7FFFFFFF))
    )
    o_ref[0] = jnp.where(bits >= lo, xv, 0.0)


def kernel(x):
    n, c, h, w = x.shape
    hw = h * w
    k = max(1, int(TOPK_FRAC * hw))
    rows = n * c
    assert hw % 128 == 0
    r = hw // 128
    xr = x.reshape(rows, r, 128)
    out = pl.pallas_call(
        functools.partial(_select_mask_kernel, k=k),
        grid=(rows,),
        in_specs=[pl.BlockSpec((1, r, 128), lambda i: (i, 0, 0))],
        out_specs=pl.BlockSpec((1, r, 128), lambda i: (i, 0, 0)),
        out_shape=jax.ShapeDtypeStruct((rows, r, 128), jnp.float32),
    )(xr)
    return out.reshape(n, c, h, w)


# speedup vs baseline: 26.4347x; 2.3585x over previous
import functools

import jax
import jax.numpy as jnp
from jax.experimental import pallas as pl

TOPK_FRAC = 0.1
ROWS_PER_BLOCK = 8


def _select_mask_kernel(x_ref, o_ref, *, k: int, n_iters: int = 31):
    xv = x_ref[...]
    bits = jax.lax.bitcast_convert_type(xv, jnp.int32) & jnp.int32(0x7FFFFFFF)
    r = xv.shape[0]

    def body(_, lohi):
        lo, hi = lohi
        mid = lo + ((hi - lo) >> 1)
        cnt = jnp.sum((bits >= mid).astype(jnp.int32), axis=1, keepdims=True)
        ge = cnt >= k
        return (jnp.where(ge, mid, lo), jnp.where(ge, hi, mid))

    lo0 = jnp.zeros((r, 1), jnp.int32)
    hi0 = jnp.full((r, 1), 0x7FFFFFFF, jnp.int32)
    lo, _ = jax.lax.fori_loop(0, n_iters, body, (lo0, hi0))
    o_ref[...] = jnp.where(bits >= lo, xv, 0.0)


def kernel(x):
    n, c, h, w = x.shape
    hw = h * w
    k = max(1, int(TOPK_FRAC * hw))
    rows = n * c
    assert rows % ROWS_PER_BLOCK == 0
    xr = x.reshape(rows, hw)
    out = pl.pallas_call(
        functools.partial(_select_mask_kernel, k=k),
        grid=(rows // ROWS_PER_BLOCK,),
        in_specs=[pl.BlockSpec((ROWS_PER_BLOCK, hw), lambda i: (i, 0))],
        out_specs=pl.BlockSpec((ROWS_PER_BLOCK, hw), lambda i: (i, 0)),
        out_shape=jax.ShapeDtypeStruct((rows, hw), jnp.float32),
    )(xr)
    return out.reshape(n, c, h, w)
